# Initial kernel scaffold; baseline (speedup 1.0000x reference)
import jax
import jax.numpy as jnp
from jax.experimental import pallas as pl


def kernel(input_ids, speaker_ids, char_emb, spk_table, fc_w, fc_b, ln_gamma, ln_beta):
    raise NotImplementedError("write your pallas kernel here")


# speedup vs baseline: 3.5393x; 3.5393x over previous
import functools

import jax
import jax.numpy as jnp
from jax import lax
from jax.experimental import pallas as pl
from jax.experimental.pallas import tpu as pltpu
from jax.experimental.pallas import tpu_sc as plsc

B, L, V, H = 1024, 200, 1000, 512
N_SPK, SPK_U = 128, 64
EPS = 1e-05

NC, NS = 2, 16
NW = NC * NS
TOKENS = B * L
PER_W = TOKENS // NW
CHUNK = 128
N_CHUNKS = PER_W // CHUNK


def _prep_body(emb_ref, g_ref, b_ref, spk_ids_ref, spk_tab_ref, fc_w_ref,
               fc_b_ref, ln_out_ref, spk_out_ref):
    x = emb_ref[...]
    mean = jnp.mean(x, axis=1, keepdims=True)
    xc = x - mean
    var = jnp.mean(xc * xc, axis=1, keepdims=True)
    ln_out_ref[...] = (xc * lax.rsqrt(var + EPS) * g_ref[0, :][None, :]
                       + b_ref[0, :][None, :])
    sid = spk_ids_ref[...]
    onehot = (sid == lax.broadcasted_iota(jnp.int32, (B, N_SPK), 1))
    se = jnp.dot(onehot.astype(jnp.float32), spk_tab_ref[...],
                 preferred_element_type=jnp.float32)
    feat = jnp.dot(se, fc_w_ref[...], preferred_element_type=jnp.float32)
    feat = feat + fc_b_ref[0, :][None, :]
    spk_out_ref[...] = jax.nn.softplus(feat)


_prep = pl.pallas_call(
    _prep_body,
    out_shape=[
        jax.ShapeDtypeStruct((V, H), jnp.float32),
        jax.ShapeDtypeStruct((B, H), jnp.float32),
    ],
)


_sc_mesh = plsc.VectorSubcoreMesh(core_axis_name="c", subcore_axis_name="s")


@functools.partial(
    pl.kernel,
    out_type=jax.ShapeDtypeStruct((TOKENS, H), jnp.float32),
    mesh=_sc_mesh,
    scratch_types=[
        pltpu.VMEM((PER_W,), jnp.int32),
        pltpu.VMEM((CHUNK, H), jnp.float32),
        pltpu.SemaphoreType.DMA,
    ],
)
def _sc_gather(table_hbm, idx_hbm, out_hbm, idx_v, rows_v, sem):
    wid = lax.axis_index("s") * NC + lax.axis_index("c")
    base = pl.multiple_of(wid * PER_W, PER_W)
    pltpu.sync_copy(idx_hbm.at[pl.ds(base, PER_W)], idx_v)

    @pl.loop(0, N_CHUNKS)
    def _(j):
        off = pl.multiple_of(j * CHUNK, CHUNK)
        pltpu.async_copy(table_hbm.at[idx_v.at[pl.ds(off, CHUNK)]], rows_v,
                         sem).wait()
        pltpu.sync_copy(rows_v, out_hbm.at[pl.ds(base + off, CHUNK)])


def kernel(input_ids, speaker_ids, char_emb, spk_table, fc_w, fc_b, ln_gamma,
           ln_beta):
    ln_table, spk_feat = _prep(char_emb, ln_gamma.reshape(1, H),
                               ln_beta.reshape(1, H), speaker_ids, spk_table,
                               fc_w, fc_b.reshape(1, H))
    flat = _sc_gather(ln_table, input_ids.reshape(TOKENS))
    return flat.reshape(B, L, H), spk_feat.reshape(B, 1, H)


# speedup vs baseline: 3.6518x; 1.0318x over previous
import functools

import jax
import jax.numpy as jnp
from jax import lax
from jax.experimental import pallas as pl
from jax.experimental.pallas import tpu as pltpu
from jax.experimental.pallas import tpu_sc as plsc

B, L, V, H = 1024, 200, 1000, 512
N_SPK, SPK_U = 128, 64
EPS = 1e-05

NC, NS = 2, 16
NW = NC * NS
TOKENS = B * L
PER_W = TOKENS // NW
CHUNK = 64
N_CHUNKS = PER_W // CHUNK


def _prep_body(emb_ref, g_ref, b_ref, spk_ids_ref, spk_tab_ref, fc_w_ref,
               fc_b_ref, ln_out_ref, spk_out_ref):
    x = emb_ref[...]
    mean = jnp.mean(x, axis=1, keepdims=True)
    xc = x - mean
    var = jnp.mean(xc * xc, axis=1, keepdims=True)
    ln_out_ref[...] = (xc * lax.rsqrt(var + EPS) * g_ref[0, :][None, :]
                       + b_ref[0, :][None, :])
    sid = spk_ids_ref[...]
    onehot = (sid == lax.broadcasted_iota(jnp.int32, (B, N_SPK), 1))
    se = jnp.dot(onehot.astype(jnp.float32), spk_tab_ref[...],
                 preferred_element_type=jnp.float32)
    feat = jnp.dot(se, fc_w_ref[...], preferred_element_type=jnp.float32)
    feat = feat + fc_b_ref[0, :][None, :]
    spk_out_ref[...] = jax.nn.softplus(feat)


_prep = pl.pallas_call(
    _prep_body,
    out_shape=[
        jax.ShapeDtypeStruct((V, H), jnp.float32),
        jax.ShapeDtypeStruct((B, H), jnp.float32),
    ],
)


_sc_mesh = plsc.VectorSubcoreMesh(core_axis_name="c", subcore_axis_name="s")


@functools.partial(
    pl.kernel,
    out_type=jax.ShapeDtypeStruct((TOKENS, H), jnp.float32),
    mesh=_sc_mesh,
    scratch_types=[
        pltpu.VMEM((PER_W,), jnp.int32),
        pltpu.VMEM((CHUNK, H), jnp.float32),
        pltpu.VMEM((CHUNK, H), jnp.float32),
        pltpu.SemaphoreType.DMA,
        pltpu.SemaphoreType.DMA,
        pltpu.SemaphoreType.DMA,
        pltpu.SemaphoreType.DMA,
    ],
)
def _sc_gather(table_hbm, idx_hbm, out_hbm, idx_v, buf0, buf1,
               gs0, gs1, os0, os1):
    wid = lax.axis_index("s") * NC + lax.axis_index("c")
    base = pl.multiple_of(wid * PER_W, PER_W)
    pltpu.sync_copy(idx_hbm.at[pl.ds(base, PER_W)], idx_v)

    def gat(c, buf, sem):
        off = pl.multiple_of(c * CHUNK, CHUNK)
        return pltpu.make_async_copy(
            table_hbm.at[idx_v.at[pl.ds(off, CHUNK)]], buf, sem)

    def put(c, buf, sem):
        off = pl.multiple_of(c * CHUNK, CHUNK)
        return pltpu.make_async_copy(buf, out_hbm.at[pl.ds(base + off, CHUNK)],
                                     sem)

    gat(0, buf0, gs0).start()
    gat(1, buf1, gs1).start()

    @pl.loop(0, N_CHUNKS - 2, step=2)
    def _(j):
        gat(j, buf0, gs0).wait()
        put(j, buf0, os0).start()
        gat(j + 1, buf1, gs1).wait()
        put(j + 1, buf1, os1).start()
        put(j, buf0, os0).wait()
        gat(j + 2, buf0, gs0).start()
        put(j + 1, buf1, os1).wait()
        gat(j + 3, buf1, gs1).start()

    gat(N_CHUNKS - 2, buf0, gs0).wait()
    put(N_CHUNKS - 2, buf0, os0).start()
    gat(N_CHUNKS - 1, buf1, gs1).wait()
    put(N_CHUNKS - 1, buf1, os1).start()
    put(N_CHUNKS - 2, buf0, os0).wait()
    put(N_CHUNKS - 1, buf1, os1).wait()


def kernel(input_ids, speaker_ids, char_emb, spk_table, fc_w, fc_b, ln_gamma,
           ln_beta):
    ln_table, spk_feat = _prep(char_emb, ln_gamma.reshape(1, H),
                               ln_beta.reshape(1, H), speaker_ids, spk_table,
                               fc_w, fc_b.reshape(1, H))
    flat = _sc_gather(ln_table, input_ids.reshape(TOKENS))
    return flat.reshape(B, L, H), spk_feat.reshape(B, 1, H)
